# baseline (speedup 1.0000x reference)
import functools

import jax
import jax.numpy as jnp
from jax import lax
from jax.experimental import pallas as pl
from jax.experimental.pallas import tpu as pltpu
from jax.experimental.pallas import tpu_sc as plsc

_CHUNK = 128


def _fps_body(x_ref, y_ref, z_ref, idx_ref, px_ref, py_ref, pz_ref,
              dists_ref, carry_ref):
    B, N = x_ref.shape
    c = pl.program_id(0)

    X = x_ref[...]
    Y = y_ref[...]
    Z = z_ref[...]
    iota = (jax.lax.broadcasted_iota(jnp.int32, (B, N), 1)
            + N * jax.lax.broadcasted_iota(jnp.int32, (B, N), 0))
    row_base = N * jax.lax.broadcasted_iota(jnp.int32, (B, 1), 0)
    lane = jax.lax.broadcasted_iota(jnp.int32, (B, _CHUNK), 1)

    @pl.when(c == 0)
    def _init():
        dists_ref[...] = jnp.full((B, N), jnp.inf, dtype=jnp.float32)
        sx0 = jnp.broadcast_to(X[:, 0:1], (B, _CHUNK))
        sy0 = jnp.broadcast_to(Y[:, 0:1], (B, _CHUNK))
        sz0 = jnp.broadcast_to(Z[:, 0:1], (B, _CHUNK))
        carry_ref[0] = sx0
        carry_ref[1] = sy0
        carry_ref[2] = sz0

    sx = carry_ref[0][:, 0:1]
    sy = carry_ref[1][:, 0:1]
    sz = carry_ref[2][:, 0:1]
    dists = dists_ref[...]
    zero_buf = jnp.zeros((B, _CHUNK), jnp.float32)

    def step(j, st):
        dists, sx, sy, sz, bi, bx, by, bz = st
        t = c * _CHUNK + j
        dx = X - sx
        dy = Y - sy
        dz = Z - sz
        d = (dx * dx + dz * dz) + dy * dy
        dists = jnp.minimum(dists, d)
        m = jnp.max(dists, axis=1, keepdims=True)
        cand = jnp.where(dists == m, iota, B * N)
        sel = jnp.min(cand, axis=1, keepdims=True)
        sel = jnp.where(t == 0, row_base, sel)
        onehot = iota == sel
        sx = jnp.sum(jnp.where(onehot, X, 0.0), axis=1, keepdims=True)
        sy = jnp.sum(jnp.where(onehot, Y, 0.0), axis=1, keepdims=True)
        sz = jnp.sum(jnp.where(onehot, Z, 0.0), axis=1, keepdims=True)
        hit = lane == j
        bi = jnp.where(hit, sel, bi)
        bx = jnp.where(hit, sx, bx)
        by = jnp.where(hit, sy, by)
        bz = jnp.where(hit, sz, bz)
        return (dists, sx, sy, sz, bi, bx, by, bz)

    st0 = (dists, sx, sy, sz,
           jnp.zeros((B, _CHUNK), jnp.int32), zero_buf, zero_buf, zero_buf)
    dists, sx, sy, sz, bi, bx, by, bz = jax.lax.fori_loop(
        0, _CHUNK, step, st0)

    dists_ref[...] = dists
    carry_ref[0] = jnp.broadcast_to(sx, (B, _CHUNK))
    carry_ref[1] = jnp.broadcast_to(sy, (B, _CHUNK))
    carry_ref[2] = jnp.broadcast_to(sz, (B, _CHUNK))
    idx_ref[...] = bi
    px_ref[...] = bx
    py_ref[...] = by
    pz_ref[...] = bz


def _run_fps(xs, ys, zs, n_samples):
    B, N = xs.shape
    nchunk = n_samples // _CHUNK
    grid = (nchunk,)
    full = pl.BlockSpec((B, N), lambda c: (0, 0))
    out = pl.BlockSpec((B, _CHUNK), lambda c: (0, c))
    return pl.pallas_call(
        _fps_body,
        grid=grid,
        in_specs=[full, full, full],
        out_specs=[out, out, out, out],
        out_shape=[
            jax.ShapeDtypeStruct((B, n_samples), jnp.int32),
            jax.ShapeDtypeStruct((B, n_samples), jnp.float32),
            jax.ShapeDtypeStruct((B, n_samples), jnp.float32),
            jax.ShapeDtypeStruct((B, n_samples), jnp.float32),
        ],
        scratch_shapes=[
            pltpu.VMEM((B, N), jnp.float32),
            pltpu.VMEM((3, B, _CHUNK), jnp.float32),
        ],
    )(xs, ys, zs)


def _sc_gather(table, idx2d):
    V, D = table.shape
    R = idx2d.shape[0]
    info = plsc.get_sparse_core_info()
    nw = info.num_cores * info.num_subcores
    rows_per_w = R // nw

    mesh = plsc.VectorSubcoreMesh(core_axis_name="c", subcore_axis_name="s")

    @functools.partial(
        pl.kernel,
        mesh=mesh,
        compiler_params=pltpu.CompilerParams(use_tc_tiling_on_sc=False),
        out_type=jax.ShapeDtypeStruct((R * 128, D), jnp.float32),
        scratch_types=[
            pltpu.VMEM((rows_per_w, 128), jnp.int32),
            pltpu.VMEM((128, D), jnp.float32),
            pltpu.VMEM((128, D), jnp.float32),
            pltpu.SemaphoreType.DMA,
            pltpu.SemaphoreType.DMA,
        ],
    )
    def gather_kernel(table_hbm, idx_hbm, out_hbm, idx_v, rows0, rows1,
                      sem0, sem1):
        wid = lax.axis_index("s") * info.num_cores + lax.axis_index("c")
        base_row = wid * rows_per_w
        pltpu.sync_copy(idx_hbm.at[pl.ds(base_row, rows_per_w)], idx_v)
        bufs = (rows0, rows1)
        sems = (sem0, sem1)
        copies = [None, None]
        for k in range(rows_per_w):
            s = k % 2
            if copies[s] is not None:
                copies[s].wait()
            cp = pltpu.async_copy(table_hbm.at[idx_v.at[k]], bufs[s], sems[s])
            cp.wait()
            out_cp = pltpu.async_copy(
                bufs[s], out_hbm.at[pl.ds((base_row + k) * 128, 128)],
                sems[s])
            copies[s] = out_cp
        for cp in copies:
            if cp is not None:
                cp.wait()

    return gather_kernel(table, idx2d)


def kernel(points, features):
    B, N, F = points.shape[0], points.shape[1], features.shape[2]
    n_samples = N // 2
    xs = points[:, :, 0]
    ys = points[:, :, 1]
    zs = points[:, :, 2]
    idx, px, py, pz = _run_fps(xs, ys, zs, n_samples)
    downsampled_points = jnp.stack([px, py, pz], axis=-1)
    feats_flat = features.reshape(B * N, F)
    idx2d = idx.reshape(B * n_samples // 128, 128)
    downsampled_features = _sc_gather(feats_flat, idx2d).reshape(
        B, n_samples, F)
    return (downsampled_points, downsampled_features)


# speedup vs baseline: 1.1389x; 1.1389x over previous
import functools

import jax
import jax.numpy as jnp
from jax import lax
from jax.experimental import pallas as pl
from jax.experimental.pallas import tpu as pltpu
from jax.experimental.pallas import tpu_sc as plsc

_CHUNK = 128


def _fps_body(x_ref, y_ref, z_ref, idx_ref, px_ref, py_ref, pz_ref,
              dists_ref, carry_ref):
    B, N = x_ref.shape
    NT = N // 128
    c = pl.program_id(0)

    lane = jax.lax.broadcasted_iota(jnp.int32, (B, _CHUNK), 1)
    row_base = N * jax.lax.broadcasted_iota(jnp.int32, (B, 1), 0)

    @pl.when(c == 0)
    def _init():
        dists_ref[...] = jnp.full((B, N), jnp.inf, dtype=jnp.float32)
        sx0 = jnp.broadcast_to(x_ref[:, 0:1], (B, _CHUNK))
        sy0 = jnp.broadcast_to(y_ref[:, 0:1], (B, _CHUNK))
        sz0 = jnp.broadcast_to(z_ref[:, 0:1], (B, _CHUNK))
        carry_ref[0] = sx0
        carry_ref[1] = sy0
        carry_ref[2] = sz0

    sx = carry_ref[0][:, 0:1]
    sy = carry_ref[1][:, 0:1]
    sz = carry_ref[2][:, 0:1]
    zero_buf = jnp.zeros((B, _CHUNK), jnp.float32)

    def step(j, st):
        sx, sy, sz, bi, bx, by, bz = st
        t = c * _CHUNK + j

        items = []
        for k in range(NT):
            sl = pl.ds(k * 128, 128)
            xk = x_ref[:, sl]
            yk = y_ref[:, sl]
            zk = z_ref[:, sl]
            dx = xk - sx
            dy = yk - sy
            dz = zk - sz
            d = (dx * dx + dz * dz) + dy * dy
            nd = jnp.minimum(dists_ref[:, sl], d)
            dists_ref[:, sl] = nd
            items.append((nd, k, xk, yk, zk))

        while len(items) > 1:
            nxt = []
            for a, b in zip(items[0::2], items[1::2]):
                cond = b[0] > a[0]
                nxt.append(tuple(jnp.where(cond, bb, aa)
                                 for aa, bb in zip(a, b)))
            items = nxt
        mv, rv, xv, yv, zv = items[0]

        jv = (rv * 128 + lane) + row_base
        m = jnp.max(mv, axis=1, keepdims=True)
        ismax = mv == m
        sel = jnp.min(jnp.where(ismax, jv, B * N), axis=1, keepdims=True)
        win = jv == sel
        sx = jnp.sum(jnp.where(win, xv, 0.0), axis=1, keepdims=True)
        sy = jnp.sum(jnp.where(win, yv, 0.0), axis=1, keepdims=True)
        sz = jnp.sum(jnp.where(win, zv, 0.0), axis=1, keepdims=True)

        first = t == 0
        sel = jnp.where(first, row_base, sel)
        sx = jnp.where(first, x_ref[:, 0:1], sx)
        sy = jnp.where(first, y_ref[:, 0:1], sy)
        sz = jnp.where(first, z_ref[:, 0:1], sz)

        hit = lane == j
        bi = jnp.where(hit, sel, bi)
        bx = jnp.where(hit, sx, bx)
        by = jnp.where(hit, sy, by)
        bz = jnp.where(hit, sz, bz)
        return (sx, sy, sz, bi, bx, by, bz)

    st0 = (sx, sy, sz,
           jnp.zeros((B, _CHUNK), jnp.int32), zero_buf, zero_buf, zero_buf)
    sx, sy, sz, bi, bx, by, bz = jax.lax.fori_loop(0, _CHUNK, step, st0)

    carry_ref[0] = jnp.broadcast_to(sx, (B, _CHUNK))
    carry_ref[1] = jnp.broadcast_to(sy, (B, _CHUNK))
    carry_ref[2] = jnp.broadcast_to(sz, (B, _CHUNK))
    idx_ref[...] = bi
    px_ref[...] = bx
    py_ref[...] = by
    pz_ref[...] = bz


def _run_fps(xs, ys, zs, n_samples):
    B, N = xs.shape
    nchunk = n_samples // _CHUNK
    grid = (nchunk,)
    full = pl.BlockSpec((B, N), lambda c: (0, 0))
    out = pl.BlockSpec((B, _CHUNK), lambda c: (0, c))
    return pl.pallas_call(
        _fps_body,
        grid=grid,
        in_specs=[full, full, full],
        out_specs=[out, out, out, out],
        out_shape=[
            jax.ShapeDtypeStruct((B, n_samples), jnp.int32),
            jax.ShapeDtypeStruct((B, n_samples), jnp.float32),
            jax.ShapeDtypeStruct((B, n_samples), jnp.float32),
            jax.ShapeDtypeStruct((B, n_samples), jnp.float32),
        ],
        scratch_shapes=[
            pltpu.VMEM((B, N), jnp.float32),
            pltpu.VMEM((3, B, _CHUNK), jnp.float32),
        ],
    )(xs, ys, zs)


def _sc_gather(table, idx2d):
    V, D = table.shape
    R = idx2d.shape[0]
    info = plsc.get_sparse_core_info()
    nw = info.num_cores * info.num_subcores
    rows_per_w = R // nw

    mesh = plsc.VectorSubcoreMesh(core_axis_name="c", subcore_axis_name="s")

    @functools.partial(
        pl.kernel,
        mesh=mesh,
        compiler_params=pltpu.CompilerParams(use_tc_tiling_on_sc=False),
        out_type=jax.ShapeDtypeStruct((R * 128, D), jnp.float32),
        scratch_types=[
            pltpu.VMEM((rows_per_w, 128), jnp.int32),
            pltpu.VMEM((128, D), jnp.float32),
            pltpu.VMEM((128, D), jnp.float32),
            pltpu.SemaphoreType.DMA,
            pltpu.SemaphoreType.DMA,
        ],
    )
    def gather_kernel(table_hbm, idx_hbm, out_hbm, idx_v, rows0, rows1,
                      sem0, sem1):
        wid = lax.axis_index("s") * info.num_cores + lax.axis_index("c")
        base_row = wid * rows_per_w
        pltpu.sync_copy(idx_hbm.at[pl.ds(base_row, rows_per_w)], idx_v)
        bufs = (rows0, rows1)
        sems = (sem0, sem1)
        copies = [None, None]
        for k in range(rows_per_w):
            s = k % 2
            if copies[s] is not None:
                copies[s].wait()
            cp = pltpu.async_copy(table_hbm.at[idx_v.at[k]], bufs[s], sems[s])
            cp.wait()
            out_cp = pltpu.async_copy(
                bufs[s], out_hbm.at[pl.ds((base_row + k) * 128, 128)],
                sems[s])
            copies[s] = out_cp
        for cp in copies:
            if cp is not None:
                cp.wait()

    return gather_kernel(table, idx2d)


def kernel(points, features):
    B, N, F = points.shape[0], points.shape[1], features.shape[2]
    n_samples = N // 2
    xs = points[:, :, 0]
    ys = points[:, :, 1]
    zs = points[:, :, 2]
    idx, px, py, pz = _run_fps(xs, ys, zs, n_samples)
    downsampled_points = jnp.stack([px, py, pz], axis=-1)
    feats_flat = features.reshape(B * N, F)
    idx2d = idx.reshape(B * n_samples // 128, 128)
    downsampled_features = _sc_gather(feats_flat, idx2d).reshape(
        B, n_samples, F)
    return (downsampled_points, downsampled_features)


# speedup vs baseline: 1.2632x; 1.1091x over previous
import functools

import jax
import jax.numpy as jnp
from jax import lax
from jax.experimental import pallas as pl
from jax.experimental.pallas import tpu as pltpu
from jax.experimental.pallas import tpu_sc as plsc

_CHUNK = 128


def _fps_body(x_ref, y_ref, z_ref, idx_ref, px_ref, py_ref, pz_ref,
              dists_ref, carry_ref):
    B, N = x_ref.shape
    NT = N // 128
    c = pl.program_id(0)

    lane = jax.lax.broadcasted_iota(jnp.int32, (B, _CHUNK), 1)
    row_base = N * jax.lax.broadcasted_iota(jnp.int32, (B, 1), 0)

    @pl.when(c == 0)
    def _init():
        dists_ref[...] = jnp.full((B, N), jnp.inf, dtype=jnp.float32)
        sx0 = jnp.broadcast_to(x_ref[:, 0:1], (B, _CHUNK))
        sy0 = jnp.broadcast_to(y_ref[:, 0:1], (B, _CHUNK))
        sz0 = jnp.broadcast_to(z_ref[:, 0:1], (B, _CHUNK))
        carry_ref[0] = sx0
        carry_ref[1] = sy0
        carry_ref[2] = sz0

    sx = carry_ref[0][:, 0:1]
    sy = carry_ref[1][:, 0:1]
    sz = carry_ref[2][:, 0:1]
    zero_buf = jnp.zeros((B, _CHUNK), jnp.float32)

    def step(j, st):
        sx, sy, sz, bi, bx, by, bz = st
        t = c * _CHUNK + j

        NSTR = 8
        streams = [None] * NSTR
        for k in range(NT):
            sl = pl.ds(k * 128, 128)
            xk = x_ref[:, sl]
            yk = y_ref[:, sl]
            zk = z_ref[:, sl]
            dx = xk - sx
            dy = yk - sy
            dz = zk - sz
            d = (dx * dx + dz * dz) + dy * dy
            nd = jnp.minimum(dists_ref[:, sl], d)
            dists_ref[:, sl] = nd
            s = k % NSTR
            if streams[s] is None:
                streams[s] = (nd, jnp.full((B, 128), k, jnp.int32),
                              xk, yk, zk)
            else:
                a = streams[s]
                cond = nd > a[0]
                bnode = (nd, k, xk, yk, zk)
                streams[s] = tuple(jnp.where(cond, bb, aa)
                                   for aa, bb in zip(a, bnode))

        while len(streams) > 1:
            nxt = []
            for a, b in zip(streams[0::2], streams[1::2]):
                take = (b[0] > a[0]) | ((b[0] == a[0]) & (b[1] < a[1]))
                nxt.append(tuple(jnp.where(take, bb, aa)
                                 for aa, bb in zip(a, b)))
            streams = nxt
        mv, rv, xv, yv, zv = streams[0]

        jv = (rv * 128 + lane) + row_base
        m = jnp.max(mv, axis=1, keepdims=True)
        ismax = mv == m
        sel = jnp.min(jnp.where(ismax, jv, B * N), axis=1, keepdims=True)
        win = jv == sel
        sx = jnp.sum(jnp.where(win, xv, 0.0), axis=1, keepdims=True)
        sy = jnp.sum(jnp.where(win, yv, 0.0), axis=1, keepdims=True)
        sz = jnp.sum(jnp.where(win, zv, 0.0), axis=1, keepdims=True)

        first = t == 0
        sel = jnp.where(first, row_base, sel)
        sx = jnp.where(first, x_ref[:, 0:1], sx)
        sy = jnp.where(first, y_ref[:, 0:1], sy)
        sz = jnp.where(first, z_ref[:, 0:1], sz)

        hit = lane == j
        bi = jnp.where(hit, sel, bi)
        bx = jnp.where(hit, sx, bx)
        by = jnp.where(hit, sy, by)
        bz = jnp.where(hit, sz, bz)
        return (sx, sy, sz, bi, bx, by, bz)

    st0 = (sx, sy, sz,
           jnp.zeros((B, _CHUNK), jnp.int32), zero_buf, zero_buf, zero_buf)
    sx, sy, sz, bi, bx, by, bz = jax.lax.fori_loop(0, _CHUNK, step, st0)

    carry_ref[0] = jnp.broadcast_to(sx, (B, _CHUNK))
    carry_ref[1] = jnp.broadcast_to(sy, (B, _CHUNK))
    carry_ref[2] = jnp.broadcast_to(sz, (B, _CHUNK))
    idx_ref[...] = bi
    px_ref[...] = bx
    py_ref[...] = by
    pz_ref[...] = bz


def _run_fps(xs, ys, zs, n_samples):
    B, N = xs.shape
    nchunk = n_samples // _CHUNK
    grid = (nchunk,)
    full = pl.BlockSpec((B, N), lambda c: (0, 0))
    out = pl.BlockSpec((B, _CHUNK), lambda c: (0, c))
    return pl.pallas_call(
        _fps_body,
        grid=grid,
        in_specs=[full, full, full],
        out_specs=[out, out, out, out],
        out_shape=[
            jax.ShapeDtypeStruct((B, n_samples), jnp.int32),
            jax.ShapeDtypeStruct((B, n_samples), jnp.float32),
            jax.ShapeDtypeStruct((B, n_samples), jnp.float32),
            jax.ShapeDtypeStruct((B, n_samples), jnp.float32),
        ],
        scratch_shapes=[
            pltpu.VMEM((B, N), jnp.float32),
            pltpu.VMEM((3, B, _CHUNK), jnp.float32),
        ],
    )(xs, ys, zs)


def _sc_gather(table, idx2d):
    V, D = table.shape
    R = idx2d.shape[0]
    info = plsc.get_sparse_core_info()
    nw = info.num_cores * info.num_subcores
    rows_per_w = R // nw

    mesh = plsc.VectorSubcoreMesh(core_axis_name="c", subcore_axis_name="s")

    @functools.partial(
        pl.kernel,
        mesh=mesh,
        compiler_params=pltpu.CompilerParams(use_tc_tiling_on_sc=False),
        out_type=jax.ShapeDtypeStruct((R * 128, D), jnp.float32),
        scratch_types=[
            pltpu.VMEM((rows_per_w, 128), jnp.int32),
            pltpu.VMEM((128, D), jnp.float32),
            pltpu.VMEM((128, D), jnp.float32),
            pltpu.SemaphoreType.DMA,
            pltpu.SemaphoreType.DMA,
        ],
    )
    def gather_kernel(table_hbm, idx_hbm, out_hbm, idx_v, rows0, rows1,
                      sem0, sem1):
        wid = lax.axis_index("s") * info.num_cores + lax.axis_index("c")
        base_row = wid * rows_per_w
        pltpu.sync_copy(idx_hbm.at[pl.ds(base_row, rows_per_w)], idx_v)
        bufs = (rows0, rows1)
        sems = (sem0, sem1)
        copies = [None, None]
        for k in range(rows_per_w):
            s = k % 2
            if copies[s] is not None:
                copies[s].wait()
            cp = pltpu.async_copy(table_hbm.at[idx_v.at[k]], bufs[s], sems[s])
            cp.wait()
            out_cp = pltpu.async_copy(
                bufs[s], out_hbm.at[pl.ds((base_row + k) * 128, 128)],
                sems[s])
            copies[s] = out_cp
        for cp in copies:
            if cp is not None:
                cp.wait()

    return gather_kernel(table, idx2d)


def kernel(points, features):
    B, N, F = points.shape[0], points.shape[1], features.shape[2]
    n_samples = N // 2
    xs = points[:, :, 0]
    ys = points[:, :, 1]
    zs = points[:, :, 2]
    idx, px, py, pz = _run_fps(xs, ys, zs, n_samples)
    downsampled_points = jnp.stack([px, py, pz], axis=-1)
    feats_flat = features.reshape(B * N, F)
    idx2d = idx.reshape(B * n_samples // 128, 128)
    downsampled_features = _sc_gather(feats_flat, idx2d).reshape(
        B, n_samples, F)
    return (downsampled_points, downsampled_features)


# speedup vs baseline: 1.4156x; 1.1206x over previous
import functools

import jax
import jax.numpy as jnp
from jax import lax
from jax.experimental import pallas as pl
from jax.experimental.pallas import tpu as pltpu
from jax.experimental.pallas import tpu_sc as plsc

_CHUNK = 128


def _fps_body(x_ref, y_ref, z_ref, idx_ref, px_ref, py_ref, pz_ref,
              dists_ref, carry_ref, fix_ref):
    B, N = x_ref.shape
    NT = N // 128
    c = pl.program_id(0)

    lane = jax.lax.broadcasted_iota(jnp.int32, (B, _CHUNK), 1)
    row_base = N * jax.lax.broadcasted_iota(jnp.int32, (B, 1), 0)

    @pl.when(c == 0)
    def _init():
        dists_ref[...] = jnp.full((B, N), jnp.inf, dtype=jnp.float32)
        sx0 = jnp.broadcast_to(x_ref[:, 0:1], (B, _CHUNK))
        sy0 = jnp.broadcast_to(y_ref[:, 0:1], (B, _CHUNK))
        sz0 = jnp.broadcast_to(z_ref[:, 0:1], (B, _CHUNK))
        carry_ref[0] = sx0
        carry_ref[1] = sy0
        carry_ref[2] = sz0

    sx = carry_ref[0][:, 0:1]
    sy = carry_ref[1][:, 0:1]
    sz = carry_ref[2][:, 0:1]
    zero_buf = jnp.zeros((B, _CHUNK), jnp.float32)

    def step(j, st):
        sx, sy, sz, bi, bx, by, bz = st
        t = c * _CHUNK + j

        NSTR = 8
        streams = [None] * NSTR
        for k in range(NT):
            sl = pl.ds(k * 128, 128)
            xk = x_ref[:, sl]
            yk = y_ref[:, sl]
            zk = z_ref[:, sl]
            dx = xk - sx
            dy = yk - sy
            dz = zk - sz
            d = (dx * dx + dz * dz) + dy * dy
            nd = jnp.minimum(dists_ref[:, sl], d)
            dists_ref[:, sl] = nd
            s = k % NSTR
            if streams[s] is None:
                streams[s] = (nd, jnp.full((B, 128), k, jnp.int32),
                              xk, yk, zk)
            else:
                a = streams[s]
                cond = nd > a[0]
                bnode = (nd, k, xk, yk, zk)
                streams[s] = tuple(jnp.where(cond, bb, aa)
                                   for aa, bb in zip(a, bnode))

        streams = [s for s in streams if s is not None]
        while len(streams) > 1:
            nxt = []
            for a, b in zip(streams[0::2], streams[1::2]):
                take = (b[0] > a[0]) | ((b[0] == a[0]) & (b[1] < a[1]))
                nxt.append(tuple(jnp.where(take, bb, aa)
                                 for aa, bb in zip(a, b)))
            if len(streams) % 2:
                nxt.append(streams[-1])
            streams = nxt
        mv, rv, xv, yv, zv = streams[0]

        jv = (rv * 128 + lane) + row_base
        m = jnp.max(mv, axis=1, keepdims=True)
        ismax = mv == m
        sel = jnp.min(jnp.where(ismax, jv, B * N), axis=1, keepdims=True)
        cnt = jnp.sum(jnp.where(ismax, 1.0, 0.0))
        fix_ref[0, :, 0:1] = jnp.sum(jnp.where(ismax, xv, 0.0), axis=1,
                                     keepdims=True)
        fix_ref[1, :, 0:1] = jnp.sum(jnp.where(ismax, yv, 0.0), axis=1,
                                     keepdims=True)
        fix_ref[2, :, 0:1] = jnp.sum(jnp.where(ismax, zv, 0.0), axis=1,
                                     keepdims=True)

        @pl.when(cnt > B + 0.5)
        def _fix_tie():
            win = jv == sel
            fix_ref[0, :, 0:1] = jnp.sum(jnp.where(win, xv, 0.0), axis=1,
                                         keepdims=True)
            fix_ref[1, :, 0:1] = jnp.sum(jnp.where(win, yv, 0.0), axis=1,
                                         keepdims=True)
            fix_ref[2, :, 0:1] = jnp.sum(jnp.where(win, zv, 0.0), axis=1,
                                         keepdims=True)

        sx = fix_ref[0, :, 0:1]
        sy = fix_ref[1, :, 0:1]
        sz = fix_ref[2, :, 0:1]

        first = t == 0
        sel = jnp.where(first, row_base, sel)
        sx = jnp.where(first, x_ref[:, 0:1], sx)
        sy = jnp.where(first, y_ref[:, 0:1], sy)
        sz = jnp.where(first, z_ref[:, 0:1], sz)

        hit = lane == j
        bi = jnp.where(hit, sel, bi)
        bx = jnp.where(hit, sx, bx)
        by = jnp.where(hit, sy, by)
        bz = jnp.where(hit, sz, bz)
        return (sx, sy, sz, bi, bx, by, bz)

    st0 = (sx, sy, sz,
           jnp.zeros((B, _CHUNK), jnp.int32), zero_buf, zero_buf, zero_buf)
    sx, sy, sz, bi, bx, by, bz = jax.lax.fori_loop(0, _CHUNK, step, st0)

    carry_ref[0] = jnp.broadcast_to(sx, (B, _CHUNK))
    carry_ref[1] = jnp.broadcast_to(sy, (B, _CHUNK))
    carry_ref[2] = jnp.broadcast_to(sz, (B, _CHUNK))
    idx_ref[...] = bi
    px_ref[...] = bx
    py_ref[...] = by
    pz_ref[...] = bz


def _run_fps(xs, ys, zs, n_samples):
    B, N = xs.shape
    nchunk = n_samples // _CHUNK
    grid = (nchunk,)
    full = pl.BlockSpec((B, N), lambda c: (0, 0))
    out = pl.BlockSpec((B, _CHUNK), lambda c: (0, c))
    return pl.pallas_call(
        _fps_body,
        grid=grid,
        in_specs=[full, full, full],
        out_specs=[out, out, out, out],
        out_shape=[
            jax.ShapeDtypeStruct((B, n_samples), jnp.int32),
            jax.ShapeDtypeStruct((B, n_samples), jnp.float32),
            jax.ShapeDtypeStruct((B, n_samples), jnp.float32),
            jax.ShapeDtypeStruct((B, n_samples), jnp.float32),
        ],
        scratch_shapes=[
            pltpu.VMEM((B, N), jnp.float32),
            pltpu.VMEM((3, B, _CHUNK), jnp.float32),
            pltpu.VMEM((3, B, _CHUNK), jnp.float32),
        ],
    )(xs, ys, zs)


def _sc_gather(table, idx2d):
    V, D = table.shape
    R = idx2d.shape[0]
    info = plsc.get_sparse_core_info()
    nw = info.num_cores * info.num_subcores
    rows_per_w = R // nw

    mesh = plsc.VectorSubcoreMesh(core_axis_name="c", subcore_axis_name="s")

    @functools.partial(
        pl.kernel,
        mesh=mesh,
        compiler_params=pltpu.CompilerParams(use_tc_tiling_on_sc=False),
        out_type=jax.ShapeDtypeStruct((R * 128, D), jnp.float32),
        scratch_types=[
            pltpu.VMEM((rows_per_w, 128), jnp.int32),
            pltpu.VMEM((128, D), jnp.float32),
            pltpu.VMEM((128, D), jnp.float32),
            pltpu.SemaphoreType.DMA,
            pltpu.SemaphoreType.DMA,
        ],
    )
    def gather_kernel(table_hbm, idx_hbm, out_hbm, idx_v, rows0, rows1,
                      sem0, sem1):
        wid = lax.axis_index("s") * info.num_cores + lax.axis_index("c")
        base_row = wid * rows_per_w
        pltpu.sync_copy(idx_hbm.at[pl.ds(base_row, rows_per_w)], idx_v)
        bufs = (rows0, rows1)
        sems = (sem0, sem1)
        copies = [None, None]
        for k in range(rows_per_w):
            s = k % 2
            if copies[s] is not None:
                copies[s].wait()
            cp = pltpu.async_copy(table_hbm.at[idx_v.at[k]], bufs[s], sems[s])
            cp.wait()
            out_cp = pltpu.async_copy(
                bufs[s], out_hbm.at[pl.ds((base_row + k) * 128, 128)],
                sems[s])
            copies[s] = out_cp
        for cp in copies:
            if cp is not None:
                cp.wait()

    return gather_kernel(table, idx2d)


def kernel(points, features):
    B, N, F = points.shape[0], points.shape[1], features.shape[2]
    n_samples = N // 2
    xs = points[:, :, 0]
    ys = points[:, :, 1]
    zs = points[:, :, 2]
    idx, px, py, pz = _run_fps(xs, ys, zs, n_samples)
    downsampled_points = jnp.stack([px, py, pz], axis=-1)
    feats_flat = features.reshape(B * N, F)
    idx2d = idx.reshape(B * n_samples // 128, 128)
    downsampled_features = _sc_gather(feats_flat, idx2d).reshape(
        B, n_samples, F)
    return (downsampled_points, downsampled_features)


# speedup vs baseline: 1.8172x; 1.2838x over previous
import functools

import jax
import jax.numpy as jnp
from jax import lax
from jax.experimental import pallas as pl
from jax.experimental.pallas import tpu as pltpu
from jax.experimental.pallas import tpu_sc as plsc

_CHUNK = 128


def _fps_body(x_ref, y_ref, z_ref, idx_ref, px_ref, py_ref, pz_ref,
              dists_ref, carry_ref, fix_ref):
    B, N = x_ref.shape
    NT = N // 128
    c = pl.program_id(0)

    lane = jax.lax.broadcasted_iota(jnp.int32, (B, _CHUNK), 1)
    row_base = N * jax.lax.broadcasted_iota(jnp.int32, (B, _CHUNK), 0)

    @pl.when(c == 0)
    def _init():
        dists_ref[...] = jnp.full((B, N), jnp.inf, dtype=jnp.float32)
        sx0 = jnp.broadcast_to(x_ref[:, 0:1], (B, _CHUNK))
        sy0 = jnp.broadcast_to(y_ref[:, 0:1], (B, _CHUNK))
        sz0 = jnp.broadcast_to(z_ref[:, 0:1], (B, _CHUNK))
        carry_ref[0] = sx0
        carry_ref[1] = sy0
        carry_ref[2] = sz0

    sx = carry_ref[0][...]
    sy = carry_ref[1][...]
    sz = carry_ref[2][...]
    x0b = jnp.broadcast_to(x_ref[:, 0:1], (B, _CHUNK))
    y0b = jnp.broadcast_to(y_ref[:, 0:1], (B, _CHUNK))
    z0b = jnp.broadcast_to(z_ref[:, 0:1], (B, _CHUNK))
    zero_buf = jnp.zeros((B, _CHUNK), jnp.float32)

    def step(j, st):
        sx, sy, sz, bi, bx, by, bz = st
        t = c * _CHUNK + j

        NSTR = 8
        streams = [None] * NSTR
        for k in range(NT):
            sl = pl.ds(k * 128, 128)
            xk = x_ref[:, sl]
            yk = y_ref[:, sl]
            zk = z_ref[:, sl]
            dx = xk - sx
            dy = yk - sy
            dz = zk - sz
            d = (dx * dx + dz * dz) + dy * dy
            nd = jnp.minimum(dists_ref[:, sl], d)
            dists_ref[:, sl] = nd
            s = k % NSTR
            if streams[s] is None:
                streams[s] = (nd, jnp.full((B, 128), k, jnp.int32),
                              xk, yk, zk)
            else:
                a = streams[s]
                cond = nd > a[0]
                bnode = (nd, k, xk, yk, zk)
                streams[s] = tuple(jnp.where(cond, bb, aa)
                                   for aa, bb in zip(a, bnode))

        streams = [s for s in streams if s is not None]
        while len(streams) > 1:
            nxt = []
            for a, b in zip(streams[0::2], streams[1::2]):
                take = (b[0] > a[0]) | ((b[0] == a[0]) & (b[1] < a[1]))
                nxt.append(tuple(jnp.where(take, bb, aa)
                                 for aa, bb in zip(a, b)))
            if len(streams) % 2:
                nxt.append(streams[-1])
            streams = nxt
        mv, rv, xv, yv, zv = streams[0]

        jv = (rv * 128 + lane) + row_base

        def bsum(v):
            return jnp.broadcast_to(
                jnp.sum(v, axis=1, keepdims=True), (B, _CHUNK))

        m = jnp.max(mv, axis=1, keepdims=True)
        ismax = mv == m
        sel = jnp.broadcast_to(
            jnp.min(jnp.where(ismax, jv, B * N), axis=1, keepdims=True),
            (B, _CHUNK))
        cnt = jnp.sum(jnp.where(ismax, 1.0, 0.0))
        fix_ref[0] = bsum(jnp.where(ismax, xv, 0.0))
        fix_ref[1] = bsum(jnp.where(ismax, yv, 0.0))
        fix_ref[2] = bsum(jnp.where(ismax, zv, 0.0))

        @pl.when(cnt > B + 0.5)
        def _fix_tie():
            win = jv == sel
            fix_ref[0] = bsum(jnp.where(win, xv, 0.0))
            fix_ref[1] = bsum(jnp.where(win, yv, 0.0))
            fix_ref[2] = bsum(jnp.where(win, zv, 0.0))

        sx = fix_ref[0][...]
        sy = fix_ref[1][...]
        sz = fix_ref[2][...]

        first = t == 0
        sel = jnp.where(first, row_base, sel)
        sx = jnp.where(first, x0b, sx)
        sy = jnp.where(first, y0b, sy)
        sz = jnp.where(first, z0b, sz)

        hit = lane == j
        bi = jnp.where(hit, sel, bi)
        bx = jnp.where(hit, sx, bx)
        by = jnp.where(hit, sy, by)
        bz = jnp.where(hit, sz, bz)
        return (sx, sy, sz, bi, bx, by, bz)

    st0 = (sx, sy, sz,
           jnp.zeros((B, _CHUNK), jnp.int32), zero_buf, zero_buf, zero_buf)
    sx, sy, sz, bi, bx, by, bz = jax.lax.fori_loop(0, _CHUNK, step, st0)

    carry_ref[0] = sx
    carry_ref[1] = sy
    carry_ref[2] = sz
    idx_ref[...] = bi
    px_ref[...] = bx
    py_ref[...] = by
    pz_ref[...] = bz


def _run_fps(xs, ys, zs, n_samples):
    B, N = xs.shape
    nchunk = n_samples // _CHUNK
    grid = (nchunk,)
    full = pl.BlockSpec((B, N), lambda c: (0, 0))
    out = pl.BlockSpec((B, _CHUNK), lambda c: (0, c))
    return pl.pallas_call(
        _fps_body,
        grid=grid,
        in_specs=[full, full, full],
        out_specs=[out, out, out, out],
        out_shape=[
            jax.ShapeDtypeStruct((B, n_samples), jnp.int32),
            jax.ShapeDtypeStruct((B, n_samples), jnp.float32),
            jax.ShapeDtypeStruct((B, n_samples), jnp.float32),
            jax.ShapeDtypeStruct((B, n_samples), jnp.float32),
        ],
        scratch_shapes=[
            pltpu.VMEM((B, N), jnp.float32),
            pltpu.VMEM((3, B, _CHUNK), jnp.float32),
            pltpu.VMEM((3, B, _CHUNK), jnp.float32),
        ],
    )(xs, ys, zs)


def _sc_gather(table, idx2d):
    V, D = table.shape
    R = idx2d.shape[0]
    info = plsc.get_sparse_core_info()
    nw = info.num_cores * info.num_subcores
    rows_per_w = R // nw

    mesh = plsc.VectorSubcoreMesh(core_axis_name="c", subcore_axis_name="s")

    @functools.partial(
        pl.kernel,
        mesh=mesh,
        compiler_params=pltpu.CompilerParams(use_tc_tiling_on_sc=False),
        out_type=jax.ShapeDtypeStruct((R * 128, D), jnp.float32),
        scratch_types=[
            pltpu.VMEM((rows_per_w, 128), jnp.int32),
            pltpu.VMEM((128, D), jnp.float32),
            pltpu.VMEM((128, D), jnp.float32),
            pltpu.SemaphoreType.DMA,
            pltpu.SemaphoreType.DMA,
        ],
    )
    def gather_kernel(table_hbm, idx_hbm, out_hbm, idx_v, rows0, rows1,
                      sem0, sem1):
        wid = lax.axis_index("s") * info.num_cores + lax.axis_index("c")
        base_row = wid * rows_per_w
        pltpu.sync_copy(idx_hbm.at[pl.ds(base_row, rows_per_w)], idx_v)
        bufs = (rows0, rows1)
        sems = (sem0, sem1)
        copies = [None, None]
        for k in range(rows_per_w):
            s = k % 2
            if copies[s] is not None:
                copies[s].wait()
            cp = pltpu.async_copy(table_hbm.at[idx_v.at[k]], bufs[s], sems[s])
            cp.wait()
            out_cp = pltpu.async_copy(
                bufs[s], out_hbm.at[pl.ds((base_row + k) * 128, 128)],
                sems[s])
            copies[s] = out_cp
        for cp in copies:
            if cp is not None:
                cp.wait()

    return gather_kernel(table, idx2d)


def kernel(points, features):
    B, N, F = points.shape[0], points.shape[1], features.shape[2]
    n_samples = N // 2
    xs = points[:, :, 0]
    ys = points[:, :, 1]
    zs = points[:, :, 2]
    idx, px, py, pz = _run_fps(xs, ys, zs, n_samples)
    downsampled_points = jnp.stack([px, py, pz], axis=-1)
    feats_flat = features.reshape(B * N, F)
    idx2d = idx.reshape(B * n_samples // 128, 128)
    downsampled_features = _sc_gather(feats_flat, idx2d).reshape(
        B, n_samples, F)
    return (downsampled_points, downsampled_features)
